# baseline (speedup 1.0000x reference)
import functools

import jax
import jax.numpy as jnp
from jax import lax
from jax.experimental import pallas as pl
from jax.experimental.pallas import tpu as pltpu
from jax.experimental.pallas import tpu_sc as plsc

N_FACTORS = 32
BATCH = 16384
NC = 2
NS = 16
NW = NC * NS
BPW = BATCH // NW
CHUNK = 128
NCHUNKS = BPW // CHUNK

_mesh = plsc.VectorSubcoreMesh(core_axis_name="c", subcore_axis_name="s")


@functools.partial(
    pl.kernel,
    out_type=jax.ShapeDtypeStruct((BATCH,), jnp.float32),
    mesh=_mesh,
    compiler_params=pltpu.CompilerParams(
        needs_layout_passes=False, use_tc_tiling_on_sc=False
    ),
    scratch_types=[
        pltpu.VMEM((NCHUNKS, CHUNK), jnp.int32),
        pltpu.VMEM((NCHUNKS, CHUNK), jnp.int32),
        pltpu.VMEM((CHUNK, N_FACTORS), jnp.float32),
        pltpu.VMEM((CHUNK, N_FACTORS), jnp.float32),
        pltpu.VMEM((BPW,), jnp.float32),
        pltpu.SemaphoreType.DMA,
        pltpu.SemaphoreType.DMA,
    ],
)
def _sc_dot_kernel(users_hbm, items_hbm, uf_hbm, vf_hbm, out_hbm,
                   idx_u, idx_i, ubuf, vbuf, outv, sem_u, sem_v):
    wid = lax.axis_index("s") * NC + lax.axis_index("c")
    pltpu.sync_copy(users_hbm.at[pl.ds(wid * NCHUNKS, NCHUNKS)], idx_u)
    pltpu.sync_copy(items_hbm.at[pl.ds(wid * NCHUNKS, NCHUNKS)], idx_i)
    lane = lax.iota(jnp.int32, 16)
    last_lane = lane == 15
    for c in range(NCHUNKS):
        cu = pltpu.async_copy(uf_hbm.at[idx_u.at[c]], ubuf, sem_u)
        cv = pltpu.async_copy(vf_hbm.at[idx_i.at[c]], vbuf, sem_v)
        cu.wait()
        cv.wait()

        def r_body(r, _, _c=c):
            u0 = ubuf[r, pl.ds(0, 16)]
            u1 = ubuf[r, pl.ds(16, 16)]
            v0 = vbuf[r, pl.ds(0, 16)]
            v1 = vbuf[r, pl.ds(16, 16)]
            s = plsc.cumsum(u0 * v0 + u1 * v1)
            pos = jnp.full((16,), _c * CHUNK, jnp.int32) + r
            plsc.store_scatter(outv, [pos], s, mask=last_lane)
            return 0

        lax.fori_loop(0, CHUNK, r_body, 0)
    pltpu.sync_copy(outv, out_hbm.at[pl.ds(wid * BPW, BPW)])


def kernel(data, user_factors, item_factors):
    users = data[:, 0].astype(jnp.int32).reshape(NW * NCHUNKS, CHUNK)
    items = data[:, 1].astype(jnp.int32).reshape(NW * NCHUNKS, CHUNK)
    return _sc_dot_kernel(users, items, user_factors, item_factors)
